# Initial kernel scaffold; baseline (speedup 1.0000x reference)
import jax
import jax.numpy as jnp
from jax.experimental import pallas as pl


def kernel(query, key, W0, b0, W1, b1):
    raise NotImplementedError("write your pallas kernel here")


# speedup vs baseline: 1.5181x; 1.5181x over previous
import functools

import jax
import jax.numpy as jnp
from jax import lax
from jax.experimental import pallas as pl
from jax.experimental.pallas import tpu as pltpu
from jax.experimental.pallas import tpu_sc as plsc

_D = 2048
_H = 8192
_N = 8192
_K = 6
_BH = 512
_BN = 512
_L = 16


def _v_body(q_ref, b0_ref, w0_ref, w1_ref, v_ref):
    g = pl.program_id(0)
    qt = lax.dot_general(
        w0_ref[...], q_ref[...], (((1,), (0,)), ((), ())),
        precision=lax.Precision.HIGHEST, preferred_element_type=jnp.float32)
    qt = qt + b0_ref[...]
    part = lax.dot_general(
        qt, w1_ref[...], (((0,), (0,)), ((), ())),
        precision=lax.Precision.HIGHEST, preferred_element_type=jnp.float32)

    @pl.when(g == 0)
    def _init():
        v_ref[...] = jnp.zeros_like(v_ref)

    v_ref[...] += part


def _s_body(v_ref, key_ref, s_ref):
    s_ref[...] = lax.dot_general(
        key_ref[...], v_ref[...], (((1,), (0,)), ((), ())),
        precision=lax.Precision.HIGHEST, preferred_element_type=jnp.float32)


def _topk_body(s_hbm, key_hbm, out_hbm, s_v, idx_v, rows_v, out_v, sem):
    cid = lax.axis_index("c")
    sid = lax.axis_index("s")

    @pl.when(jnp.logical_and(cid == 0, sid == 0))
    def _():
        pltpu.sync_copy(s_hbm, s_v)
        found = []
        for _t in range(_K):
            def chunk(i, carry, found=tuple(found)):
                best, bestidx = carry
                vals = s_v[pl.ds(i * _L, _L)]
                lin = i * _L + lax.iota(jnp.int32, _L)
                for fj in found:
                    vals = jnp.where(lin == fj, jnp.float32(-100000.0), vals)
                m = vals > best
                return jnp.where(m, vals, best), jnp.where(m, lin, bestidx)

            best0 = jnp.full((_L,), -jnp.inf, jnp.float32)
            idx0 = jnp.zeros((_L,), jnp.int32)
            best, bestidx = lax.fori_loop(0, _N // _L, chunk, (best0, idx0))
            gb = jnp.float32(-jnp.inf)
            gi = jnp.int32(2**30)
            for l in range(_L):
                b = best[l]
                ix = bestidx[l]
                better = (b > gb) | ((b == gb) & (ix < gi))
                gb = jnp.where(better, b, gb)
                gi = jnp.where(better, ix, gi)
            found.append(gi)

        iv = jnp.zeros((_L,), jnp.int32)
        lanes = lax.iota(jnp.int32, _L)
        for j, fj in enumerate(found):
            iv = jnp.where(lanes == j, fj, iv)
        idx_v[...] = iv
        pltpu.async_copy(key_hbm.at[idx_v], rows_v, sem).wait()

        def mean_chunk(d, _):
            acc = rows_v[0, pl.ds(d * _L, _L)]
            for j in range(1, _K):
                acc = acc + rows_v[j, pl.ds(d * _L, _L)]
            out_v[pl.ds(d * _L, _L)] = acc * jnp.float32(1.0 / _K)
            return 0

        lax.fori_loop(0, _D // _L, mean_chunk, 0)
        pltpu.sync_copy(out_v, out_hbm)


@functools.cache
def _topk_mean():
    return pl.kernel(
        _topk_body,
        out_type=jax.ShapeDtypeStruct((_D,), jnp.float32),
        mesh=plsc.VectorSubcoreMesh(core_axis_name="c", subcore_axis_name="s"),
        scratch_types=[
            pltpu.VMEM((_N,), jnp.float32),
            pltpu.VMEM((_L,), jnp.int32),
            pltpu.VMEM((_L, _D), jnp.float32),
            pltpu.VMEM((_D,), jnp.float32),
            pltpu.SemaphoreType.DMA,
        ],
    )


def kernel(query, key, W0, b0, W1, b1):
    del b1
    qcol = query.reshape(_D, 1)
    b0col = b0.reshape(_H, 1)

    v = pl.pallas_call(
        _v_body,
        grid=(_H // _BH,),
        in_specs=[
            pl.BlockSpec((_D, 1), lambda g: (0, 0)),
            pl.BlockSpec((_BH, 1), lambda g: (g, 0)),
            pl.BlockSpec((_BH, _D), lambda g: (g, 0)),
            pl.BlockSpec((_BH, _D), lambda g: (g, 0)),
        ],
        out_specs=pl.BlockSpec((1, _D), lambda g: (0, 0)),
        out_shape=jax.ShapeDtypeStruct((1, _D), jnp.float32),
    )(qcol, b0col, W0, W1)

    s = pl.pallas_call(
        _s_body,
        grid=(_N // _BN,),
        in_specs=[
            pl.BlockSpec((_D, 1), lambda g: (0, 0)),
            pl.BlockSpec((_BN, _D), lambda g: (g, 0)),
        ],
        out_specs=pl.BlockSpec((_BN, 1), lambda g: (g, 0)),
        out_shape=jax.ShapeDtypeStruct((_N, 1), jnp.float32),
    )(v.reshape(_D, 1), key)

    return _topk_mean()(s.reshape(_N), key)


# speedup vs baseline: 3.4637x; 2.2816x over previous
import functools

import jax
import jax.numpy as jnp
from jax import lax
from jax.experimental import pallas as pl
from jax.experimental.pallas import tpu as pltpu
from jax.experimental.pallas import tpu_sc as plsc

_D = 2048
_H = 8192
_N = 8192
_K = 6
_BH = 1024
_BN = 1024
_L = 16


def _v_body(q_ref, b0_ref, w0_ref, w1_ref, v_ref):
    g = pl.program_id(0)
    qt = jnp.sum(w0_ref[...] * q_ref[...], axis=1, keepdims=True) + b0_ref[...]
    part = jnp.sum(w1_ref[...] * qt, axis=0, keepdims=True)

    @pl.when(g == 0)
    def _init():
        v_ref[...] = jnp.zeros_like(v_ref)

    v_ref[...] += part


def _s_body(v_ref, key_ref, s_ref):
    s_ref[...] = jnp.sum(key_ref[...] * v_ref[...], axis=1, keepdims=True)


def _topk_body(s_hbm, key_hbm, out_hbm, s_v, idx_v, rows_v, out_v, sem):
    cid = lax.axis_index("c")
    sid = lax.axis_index("s")

    @pl.when(jnp.logical_and(cid == 0, sid == 0))
    def _():
        pltpu.sync_copy(s_hbm, s_v)
        found = []
        for _t in range(_K):
            def chunk(i, carry, found=tuple(found)):
                best, bestidx = carry
                vals = s_v[pl.ds(i * _L, _L)]
                lin = i * _L + lax.iota(jnp.int32, _L)
                for fj in found:
                    vals = jnp.where(lin == fj, jnp.float32(-100000.0), vals)
                m = vals > best
                return jnp.where(m, vals, best), jnp.where(m, lin, bestidx)

            best0 = jnp.full((_L,), -jnp.inf, jnp.float32)
            idx0 = jnp.zeros((_L,), jnp.int32)
            best, bestidx = lax.fori_loop(0, _N // _L, chunk, (best0, idx0))
            gb = jnp.float32(-jnp.inf)
            gi = jnp.int32(2**30)
            for l in range(_L):
                b = best[l]
                ix = bestidx[l]
                better = (b > gb) | ((b == gb) & (ix < gi))
                gb = jnp.where(better, b, gb)
                gi = jnp.where(better, ix, gi)
            found.append(gi)

        iv = jnp.zeros((_L,), jnp.int32)
        lanes = lax.iota(jnp.int32, _L)
        for j, fj in enumerate(found):
            iv = jnp.where(lanes == j, fj, iv)
        idx_v[...] = iv
        pltpu.async_copy(key_hbm.at[idx_v], rows_v, sem).wait()

        def mean_chunk(d, _):
            acc = rows_v[0, pl.ds(d * _L, _L)]
            for j in range(1, _K):
                acc = acc + rows_v[j, pl.ds(d * _L, _L)]
            out_v[pl.ds(d * _L, _L)] = acc * jnp.float32(1.0 / _K)
            return 0

        lax.fori_loop(0, _D // _L, mean_chunk, 0)
        pltpu.sync_copy(out_v, out_hbm)


@functools.cache
def _topk_mean():
    return pl.kernel(
        _topk_body,
        out_type=jax.ShapeDtypeStruct((_D,), jnp.float32),
        mesh=plsc.VectorSubcoreMesh(core_axis_name="c", subcore_axis_name="s"),
        scratch_types=[
            pltpu.VMEM((_N,), jnp.float32),
            pltpu.VMEM((_L,), jnp.int32),
            pltpu.VMEM((_L, _D), jnp.float32),
            pltpu.VMEM((_D,), jnp.float32),
            pltpu.SemaphoreType.DMA,
        ],
    )


def kernel(query, key, W0, b0, W1, b1):
    del b1
    qrow = query.reshape(1, _D)
    b0col = b0.reshape(_H, 1)

    v = pl.pallas_call(
        _v_body,
        grid=(_H // _BH,),
        in_specs=[
            pl.BlockSpec((1, _D), lambda g: (0, 0)),
            pl.BlockSpec((_BH, 1), lambda g: (g, 0)),
            pl.BlockSpec((_BH, _D), lambda g: (g, 0)),
            pl.BlockSpec((_BH, _D), lambda g: (g, 0)),
        ],
        out_specs=pl.BlockSpec((1, _D), lambda g: (0, 0)),
        out_shape=jax.ShapeDtypeStruct((1, _D), jnp.float32),
    )(qrow, b0col, W0, W1)

    s = pl.pallas_call(
        _s_body,
        grid=(_N // _BN,),
        in_specs=[
            pl.BlockSpec((1, _D), lambda g: (0, 0)),
            pl.BlockSpec((_BN, _D), lambda g: (g, 0)),
        ],
        out_specs=pl.BlockSpec((_BN, 1), lambda g: (g, 0)),
        out_shape=jax.ShapeDtypeStruct((_N, 1), jnp.float32),
    )(v, key)

    return _topk_mean()(s.reshape(_N), key)
